# baseline (speedup 1.0000x reference)
import jax
import jax.numpy as jnp
from jax import lax
from jax.experimental import pallas as pl
from jax.experimental.pallas import tpu as pltpu
from jax.experimental.pallas import tpu_sc as plsc

_K1 = 6
_BATCH = 16384
_C = 1000


_BLK = 256
_SPLIT = 8192

def _tc_body(prob_ref, lab_ref, out_ref):
    i = pl.program_id(0)
    x = prob_ref[...]
    lab = lab_ref[0, 0, :]
    neg = jnp.float32(-jnp.inf)

    m = jnp.max(x, axis=1)
    cnt = jnp.sum((x == m[:, None]).astype(jnp.int32), axis=1)
    ans = m
    pcnt = jnp.zeros_like(cnt)
    done = cnt >= _K1
    for _ in range(_K1 - 1):
        nm = jnp.max(jnp.where(x < m[:, None], x, neg), axis=1)
        cq = jnp.sum((x == nm[:, None]).astype(jnp.int32), axis=1)
        pcnt = jnp.where(done, pcnt, cnt)
        ans = jnp.where(done, ans, nm)
        m = jnp.where(done, m, nm)
        cnt = cnt + jnp.where(done, 0, cq)
        done = cnt >= _K1

    thresh = ans
    iota = jax.lax.broadcasted_iota(jnp.int32, (_BLK, _C), 1)
    lab_val = jnp.max(jnp.where(iota == lab[:, None], x, neg), axis=1)

    inter = (lab_val > thresh).astype(jnp.int32)
    union = pcnt + 1 - inter
    iou = inter.astype(jnp.float32) / union.astype(jnp.float32)
    part = jnp.sum(iou)

    @pl.when(i == 0)
    def _init():
        out_ref[...] = jnp.zeros((1, 1), jnp.float32)

    out_ref[...] = out_ref[...] + part


def _tc_half(prob, label):
    nb = _SPLIT // _BLK
    lab3 = label[:_SPLIT].reshape(nb, 1, _BLK)
    out = pl.pallas_call(
        _tc_body,
        grid=(nb,),
        in_specs=[
            pl.BlockSpec((_BLK, _C), lambda i: (i, 0)),
            pl.BlockSpec((1, 1, _BLK), lambda i: (i, 0, 0)),
        ],
        out_specs=pl.BlockSpec((1, 1), lambda i: (0, 0)),
        out_shape=jax.ShapeDtypeStruct((1, 1), jnp.float32),
    )(prob, lab3)
    return out[0, 0]


_NW = 32
_SC_ROWS = _BATCH - _SPLIT
_RPW = _SC_ROWS // _NW
_RB = 32
_NBAT = _RPW // _RB
_STRIDE = 1008
_NCH = _STRIDE // 16


def _sort(v, desc):
    s, _ = plsc.sort_key_val(v, v, descending=desc)
    return s


def _sc_body(prob_hbm, lab_hbm, out_hbm, buf0, buf1, labv, outv, sem0, sem1):
    cid = lax.axis_index("c")
    sid = lax.axis_index("s")
    wid = sid * 2 + cid
    row0 = wid * _RPW

    neg = jnp.full((16,), -jnp.inf, jnp.float32)
    lane = lax.broadcasted_iota(jnp.int32, (16,), 0)
    five = jnp.full((16,), _K1 - 1, jnp.int32)
    ones = jnp.ones((16,), jnp.int32)
    zeros = jnp.zeros((16,), jnp.int32)

    for r in range(_RB):
        buf0[r, pl.ds(992, 16)] = neg
        buf1[r, pl.ds(992, 16)] = neg

    pltpu.sync_copy(lab_hbm.at[pl.ds(row0, _RPW)], labv)

    def _issue(b, buf, sem):
        rbase = row0 + b * _RB
        for r in range(_RB):
            pltpu.async_copy(prob_hbm.at[rbase + r, :],
                             buf.at[r, pl.ds(0, _C)], sem)

    def _drain(b, buf, sem):
        rbase = row0 + b * _RB
        for r in range(_RB):
            pltpu.make_async_copy(prob_hbm.at[rbase + r, :],
                                  buf.at[r, pl.ds(0, _C)], sem).wait()

    def _top16(buf, r):
        def build(lo, width, desc):
            if lo >= _NCH:
                return neg
            if width == 1:
                return _sort(buf[r, pl.ds(lo * 16, 16)], desc)
            if lo + width // 2 >= _NCH:
                return build(lo, width // 2, desc)
            a = build(lo, width // 2, False)
            b = build(lo + width // 2, width // 2, True)
            return _sort(jnp.maximum(a, b), desc)

        return build(0, 64, True)

    def _compute(buf, b, iou_acc):
        def group_body(g, iou_acc):
            def row_body(j, carry):
                thr_vec, cnt_vec = carry
                r = g * 16 + j
                acc = _top16(buf, r)
                thr = lax.gather(
                    acc, five[:, None],
                    lax.GatherDimensionNumbers(
                        offset_dims=(), collapsed_slice_dims=(0,),
                        start_index_map=(0,)),
                    slice_sizes=(1,),
                    mode=lax.GatherScatterMode.PROMISE_IN_BOUNDS)
                cntv = plsc.all_reduce_population_count(acc > thr)
                sel = lane == j
                return (jnp.where(sel, thr, thr_vec),
                        jnp.where(sel, cntv, cnt_vec))

            thr_vec, cnt_vec = lax.fori_loop(
                0, 16, row_body,
                (jnp.zeros((16,), jnp.float32), jnp.zeros((16,), jnp.int32)))

            rows16 = g * 16 + lane
            lab16 = labv[pl.ds(b * _RB + g * 16, 16)]
            labval = plsc.load_gather(buf, [rows16, lab16])
            inter = jnp.where(labval > thr_vec, ones, zeros)
            union = cnt_vec + ones - inter
            iou = inter.astype(jnp.float32) / union.astype(jnp.float32)
            return iou_acc + iou

        return lax.fori_loop(0, _RB // 16, group_body, iou_acc)

    _issue(0, buf0, sem0)

    def super_body(i, iou_acc):
        b0 = 2 * i
        _issue(b0 + 1, buf1, sem1)
        _drain(b0, buf0, sem0)
        iou_acc = _compute(buf0, b0, iou_acc)

        @pl.when(i < _NBAT // 2 - 1)
        def _():
            _issue(b0 + 2, buf0, sem0)

        _drain(b0 + 1, buf1, sem1)
        return _compute(buf1, b0 + 1, iou_acc)

    iou_acc = lax.fori_loop(0, _NBAT // 2, super_body,
                            jnp.zeros((16,), jnp.float32))
    outv[...] = iou_acc
    pltpu.sync_copy(outv, out_hbm.at[wid])


def _sc_half(prob, label):
    mesh = plsc.VectorSubcoreMesh(core_axis_name="c", subcore_axis_name="s")
    out = pl.kernel(
        _sc_body,
        out_type=jax.ShapeDtypeStruct((_NW, 16), jnp.float32),
        mesh=mesh,
        scratch_types=[
            pltpu.VMEM((_RB, _STRIDE), jnp.float32),
            pltpu.VMEM((_RB, _STRIDE), jnp.float32),
            pltpu.VMEM((_RPW,), jnp.int32),
            pltpu.VMEM((16,), jnp.float32),
            pltpu.SemaphoreType.DMA,
            pltpu.SemaphoreType.DMA,
        ],
        compiler_params=pltpu.CompilerParams(use_tc_tiling_on_sc=False,
                                             needs_layout_passes=False,
                                             skip_device_barrier=True),
    )(lax.slice(prob, (_SPLIT, 0), (_BATCH, _C)), label[_SPLIT:])
    return jnp.sum(out)


@jax.jit
def kernel(prob, label):
    sc = _sc_half(prob, label)
    tc = _tc_half(prob, label)
    return (sc + tc) / jnp.float32(_BATCH)


# speedup vs baseline: 1.2350x; 1.2350x over previous
import jax
import jax.numpy as jnp
from jax.experimental import pallas as pl

_K1 = 6
_BATCH = 16384
_C = 1000
_BLK = 256


def _body(prob_ref, lab_ref, out_ref):
    i = pl.program_id(0)
    x = prob_ref[...]
    lab = lab_ref[0, 0, :]
    neg = jnp.float32(-jnp.inf)

    m = jnp.max(x, axis=1)
    ms = [m]
    for _ in range(_K1 - 1):
        m = jnp.max(jnp.where(x < m[:, None], x, neg), axis=1)
        ms.append(m)

    ones_n = jnp.ones((_C, 8), jnp.float32)
    rs = []
    for mj in ms:
        ind = jnp.where(x >= mj[:, None], 1.0, 0.0).astype(jnp.float32)
        rj = jax.lax.dot_general(ind, ones_n, (((1,), (0,)), ((), ())),
                                 preferred_element_type=jnp.float32)[:, 0]
        rs.append(rj)

    thresh = ms[_K1 - 1]
    pcnt = rs[_K1 - 2]
    for j in range(_K1 - 2, -1, -1):
        cond = rs[j] >= jnp.float32(_K1)
        thresh = jnp.where(cond, ms[j], thresh)
        prev = rs[j - 1] if j > 0 else jnp.zeros_like(pcnt)
        pcnt = jnp.where(cond, prev, pcnt)

    iota = jax.lax.broadcasted_iota(jnp.int32, (_BLK, _C), 1)
    lab_val = jnp.max(jnp.where(iota == lab[:, None], x, neg), axis=1)

    inter = jnp.where(lab_val > thresh, 1.0, 0.0)
    union = pcnt + 1.0 - inter
    iou = inter / union
    part = jnp.sum(iou)

    @pl.when(i == 0)
    def _init():
        out_ref[...] = jnp.zeros((1, 1), jnp.float32)

    out_ref[...] = out_ref[...] + part


@jax.jit
def kernel(prob, label):
    nb = _BATCH // _BLK
    lab3 = label.reshape(nb, 1, _BLK)
    out = pl.pallas_call(
        _body,
        grid=(nb,),
        in_specs=[
            pl.BlockSpec((_BLK, _C), lambda i: (i, 0)),
            pl.BlockSpec((1, 1, _BLK), lambda i: (i, 0, 0)),
        ],
        out_specs=pl.BlockSpec((1, 1), lambda i: (0, 0)),
        out_shape=jax.ShapeDtypeStruct((1, 1), jnp.float32),
    )(prob, lab3)
    return out[0, 0] / jnp.float32(_BATCH)


# speedup vs baseline: 1.3471x; 1.0908x over previous
import jax
import jax.numpy as jnp
from jax.experimental import pallas as pl

_K1 = 6
_BATCH = 16384
_C = 1000
_BLK = 512


def _body(prob_ref, lab_ref, out_ref):
    i = pl.program_id(0)
    x = prob_ref[...]
    lab = lab_ref[0, 0, :]
    neg = jnp.float32(-jnp.inf)

    m = jnp.max(x, axis=1)
    ms = [m]
    for _ in range(_K1 - 1):
        m = jnp.max(jnp.where(x < m[:, None], x, neg), axis=1)
        ms.append(m)

    ones_n = jnp.ones((_C, 8), jnp.float32)
    rs = []
    for mj in ms:
        ind = jnp.where(x >= mj[:, None], 1.0, 0.0).astype(jnp.float32)
        rj = jax.lax.dot_general(ind, ones_n, (((1,), (0,)), ((), ())),
                                 preferred_element_type=jnp.float32)[:, 0]
        rs.append(rj)

    thresh = ms[_K1 - 1]
    pcnt = rs[_K1 - 2]
    for j in range(_K1 - 2, -1, -1):
        cond = rs[j] >= jnp.float32(_K1)
        thresh = jnp.where(cond, ms[j], thresh)
        prev = rs[j - 1] if j > 0 else jnp.zeros_like(pcnt)
        pcnt = jnp.where(cond, prev, pcnt)

    iota = jax.lax.broadcasted_iota(jnp.int32, (_BLK, _C), 1)
    lab_val = jnp.max(jnp.where(iota == lab[:, None], x, neg), axis=1)

    inter = jnp.where(lab_val > thresh, 1.0, 0.0)
    union = pcnt + 1.0 - inter
    iou = inter / union
    part = jnp.sum(iou)

    @pl.when(i == 0)
    def _init():
        out_ref[...] = jnp.zeros((1, 1), jnp.float32)

    out_ref[...] = out_ref[...] + part


@jax.jit
def kernel(prob, label):
    nb = _BATCH // _BLK
    lab3 = label.reshape(nb, 1, _BLK)
    out = pl.pallas_call(
        _body,
        grid=(nb,),
        in_specs=[
            pl.BlockSpec((_BLK, _C), lambda i: (i, 0)),
            pl.BlockSpec((1, 1, _BLK), lambda i: (i, 0, 0)),
        ],
        out_specs=pl.BlockSpec((1, 1), lambda i: (0, 0)),
        out_shape=jax.ShapeDtypeStruct((1, 1), jnp.float32),
    )(prob, lab3)
    return out[0, 0] / jnp.float32(_BATCH)


# speedup vs baseline: 1.3674x; 1.0150x over previous
import jax
import jax.numpy as jnp
from jax.experimental import pallas as pl

_K1 = 6
_BATCH = 16384
_C = 1000
_BLK = 1024


def _body(prob_ref, lab_ref, out_ref):
    i = pl.program_id(0)
    x = prob_ref[...]
    lab = lab_ref[0, 0, :]
    neg = jnp.float32(-jnp.inf)

    m = jnp.max(x, axis=1)
    ms = [m]
    for _ in range(_K1 - 1):
        m = jnp.max(jnp.where(x < m[:, None], x, neg), axis=1)
        ms.append(m)

    ones_n = jnp.ones((_C, 8), jnp.float32)
    rs = []
    for mj in ms:
        ind = jnp.where(x >= mj[:, None], 1.0, 0.0).astype(jnp.float32)
        rj = jax.lax.dot_general(ind, ones_n, (((1,), (0,)), ((), ())),
                                 preferred_element_type=jnp.float32)[:, 0]
        rs.append(rj)

    thresh = ms[_K1 - 1]
    pcnt = rs[_K1 - 2]
    for j in range(_K1 - 2, -1, -1):
        cond = rs[j] >= jnp.float32(_K1)
        thresh = jnp.where(cond, ms[j], thresh)
        prev = rs[j - 1] if j > 0 else jnp.zeros_like(pcnt)
        pcnt = jnp.where(cond, prev, pcnt)

    iota = jax.lax.broadcasted_iota(jnp.int32, (_BLK, _C), 1)
    lab_val = jnp.max(jnp.where(iota == lab[:, None], x, neg), axis=1)

    inter = jnp.where(lab_val > thresh, 1.0, 0.0)
    union = pcnt + 1.0 - inter
    iou = inter / union
    part = jnp.sum(iou)

    @pl.when(i == 0)
    def _init():
        out_ref[...] = jnp.zeros((1, 1), jnp.float32)

    out_ref[...] = out_ref[...] + part


@jax.jit
def kernel(prob, label):
    nb = _BATCH // _BLK
    lab3 = label.reshape(nb, 1, _BLK)
    out = pl.pallas_call(
        _body,
        grid=(nb,),
        in_specs=[
            pl.BlockSpec((_BLK, _C), lambda i: (i, 0)),
            pl.BlockSpec((1, 1, _BLK), lambda i: (i, 0, 0)),
        ],
        out_specs=pl.BlockSpec((1, 1), lambda i: (0, 0)),
        out_shape=jax.ShapeDtypeStruct((1, 1), jnp.float32),
    )(prob, lab3)
    return out[0, 0] / jnp.float32(_BATCH)
